# baseline (speedup 1.0000x reference)
import functools

import jax
import jax.numpy as jnp
from jax import lax
from jax.experimental import pallas as pl
from jax.experimental.pallas import tpu as pltpu
from jax.experimental.pallas import tpu_sc as plsc

NC = 2
NS = 16
L = 16
NW = NC * NS

D = 128
B0, B1 = 4096, 200
ROWS = B0 * B1

SC_B0 = 2240
SROWS = SC_B0 * B1
TROWS = ROWS - SROWS

CHUNK = 200
ROWS_PER_W = SROWS // NW
NCHUNK = ROWS_PER_W // CHUNK
CB = CHUNK * D

BR = 512
TC_OFF = SROWS // BR


def _make_sc_permute():
  mesh = plsc.VectorSubcoreMesh(core_axis_name="c", subcore_axis_name="s")

  @functools.partial(
      pl.kernel,
      mesh=mesh,
      out_type=jax.ShapeDtypeStruct((SROWS * D,), jnp.float32),
      scratch_types=[
          pltpu.VMEM((CB,), jnp.float32),
          pltpu.VMEM((CB,), jnp.float32),
          pltpu.VMEM((CB,), jnp.float32),
          pltpu.VMEM((CB,), jnp.float32),
          pltpu.VMEM((D,), jnp.int32),
          pltpu.SemaphoreType.DMA,
          pltpu.SemaphoreType.DMA,
          pltpu.SemaphoreType.DMA,
          pltpu.SemaphoreType.DMA,
      ],
      compiler_params=pltpu.CompilerParams(needs_layout_passes=False),
  )
  def permute_kernel(x_hbm, perm_hbm, out_hbm, ib0, ib1, ob0, ob1,
                     permb, si0, si1, so0, so1):
    wid = lax.axis_index("s") * NC + lax.axis_index("c")
    base = wid * ROWS_PER_W * D

    pltpu.sync_copy(perm_hbm, permb)
    perm_vecs = [permb[pl.ds(c * L, L)] for c in range(D // L)]

    def permute_chunk(ib, ob):
      @plsc.parallel_loop(0, CHUNK, unroll=4)
      def row_body(r):
        rb = r * D
        for c in range(D // L):
          ob[pl.ds(rb + c * L, L)] = plsc.load_gather(ib, [perm_vecs[c] + rb])

    def start_in(g, ib, sem):
      pltpu.async_copy(x_hbm.at[pl.ds(base + g * CB, CB)], ib, sem)

    def start_out(g, ob, sem):
      pltpu.async_copy(ob, out_hbm.at[pl.ds(base + g * CB, CB)], sem)

    def wait_in(ib, sem):
      pltpu.make_async_copy(x_hbm.at[pl.ds(base, CB)], ib, sem).wait()

    def wait_out(ob, sem):
      pltpu.make_async_copy(ob, out_hbm.at[pl.ds(base, CB)], sem).wait()

    start_in(0, ib0, si0)
    start_in(1, ib1, si1)

    def pair_body(i, carry):
      g = i * 2

      @pl.when(i > 0)
      def _():
        wait_out(ob0, so0)

      wait_in(ib0, si0)
      permute_chunk(ib0, ob0)
      start_out(g, ob0, so0)

      @pl.when(g + 2 < NCHUNK)
      def _():
        start_in(g + 2, ib0, si0)

      @pl.when(i > 0)
      def _():
        wait_out(ob1, so1)

      wait_in(ib1, si1)
      permute_chunk(ib1, ob1)
      start_out(g + 1, ob1, so1)

      @pl.when(g + 3 < NCHUNK)
      def _():
        start_in(g + 3, ib1, si1)

      return carry

    lax.fori_loop(0, NCHUNK // 2, pair_body, 0)
    wait_out(ob0, so0)
    wait_out(ob1, so1)

  return permute_kernel


_sc_permute = _make_sc_permute()


def _tc_body(x_ref, perm_ref, o_ref):
  k_iota = lax.broadcasted_iota(jnp.int32, (D, D), 0)
  p = (k_iota == perm_ref[...]).astype(jnp.float32)
  o_ref[...] = jnp.dot(x_ref[...], p, preferred_element_type=jnp.float32)


_tc_permute = pl.pallas_call(
    _tc_body,
    grid=(TROWS // BR,),
    in_specs=[
        pl.BlockSpec((BR, D), lambda i: (TC_OFF + i, 0)),
        pl.BlockSpec((1, D), lambda i: (0, 0)),
    ],
    out_specs=pl.BlockSpec((BR, D), lambda i: (i, 0)),
    out_shape=jax.ShapeDtypeStruct((TROWS, D), jnp.float32),
)


def kernel(input, permutation):
  x_rows = input.reshape(ROWS, D)
  sc_out = _sc_permute(x_rows.reshape(ROWS * D), permutation)
  tc_out = _tc_permute(x_rows, permutation.reshape(1, D))
  return jnp.concatenate(
      [sc_out.reshape(SC_B0, B1, D), tc_out.reshape(B0 - SC_B0, B1, D)],
      axis=0)


# speedup vs baseline: 2.7070x; 2.7070x over previous
import functools

import jax
import jax.numpy as jnp
from jax import lax
from jax.experimental import pallas as pl
from jax.experimental.pallas import tpu as pltpu
from jax.experimental.pallas import tpu_sc as plsc

NC = 2
NS = 16
L = 16
NW = NC * NS

D = 128
ROWS = 4096 * 200
ROWS_PER_W = ROWS // NW
CHUNK = 100
NCHUNK = ROWS_PER_W // CHUNK
CB = CHUNK * D
NBUF = 4


def _make_sc_permute():
  mesh = plsc.VectorSubcoreMesh(core_axis_name="c", subcore_axis_name="s")

  @functools.partial(
      pl.kernel,
      mesh=mesh,
      out_type=jax.ShapeDtypeStruct((ROWS * D,), jnp.float32),
      scratch_types=(
          [pltpu.VMEM((CB,), jnp.float32) for _ in range(2 * NBUF)]
          + [pltpu.VMEM((D,), jnp.int32)]
          + [pltpu.SemaphoreType.DMA for _ in range(2 * NBUF)]
      ),
      compiler_params=pltpu.CompilerParams(needs_layout_passes=False),
  )
  def permute_kernel(x_hbm, perm_hbm, out_hbm, *scratch):
    ibufs = scratch[:NBUF]
    obufs = scratch[NBUF:2 * NBUF]
    permb = scratch[2 * NBUF]
    isems = scratch[2 * NBUF + 1:2 * NBUF + 1 + NBUF]
    osems = scratch[2 * NBUF + 1 + NBUF:]

    wid = lax.axis_index("s") * NC + lax.axis_index("c")
    base = wid * ROWS_PER_W * D

    pltpu.sync_copy(perm_hbm, permb)
    perm_vecs = [permb[pl.ds(c * L, L)] for c in range(D // L)]

    def permute_chunk(ib, ob):
      @plsc.parallel_loop(0, CHUNK, unroll=4)
      def row_body(r):
        rb = r * D
        for c in range(D // L):
          ob[pl.ds(rb + c * L, L)] = plsc.load_gather(ib, [perm_vecs[c] + rb])

    def start_in(g, b):
      pltpu.async_copy(x_hbm.at[pl.ds(base + g * CB, CB)], ibufs[b], isems[b])

    def start_out(g, b):
      pltpu.async_copy(obufs[b], out_hbm.at[pl.ds(base + g * CB, CB)],
                       osems[b])

    def wait_in(b):
      pltpu.make_async_copy(x_hbm.at[pl.ds(base, CB)], ibufs[b],
                            isems[b]).wait()

    def wait_out(b):
      pltpu.make_async_copy(obufs[b], out_hbm.at[pl.ds(base, CB)],
                            osems[b]).wait()

    for b in range(NBUF):
      start_in(b, b)

    def ring_body(i, carry):
      g = i * NBUF
      for b in range(NBUF):
        @pl.when(i > 0)
        def _():
          wait_out(b)

        wait_in(b)
        permute_chunk(ibufs[b], obufs[b])
        start_out(g + b, b)

        @pl.when(g + b + NBUF < NCHUNK)
        def _():
          start_in(g + b + NBUF, b)
      return carry

    lax.fori_loop(0, NCHUNK // NBUF, ring_body, 0)
    for b in range(NBUF):
      wait_out(b)

  return permute_kernel


_sc_permute = _make_sc_permute()


def kernel(input, permutation):
  x_flat = input.reshape(ROWS * D)
  out_flat = _sc_permute(x_flat, permutation)
  return out_flat.reshape(input.shape)


# speedup vs baseline: 2.7378x; 1.0114x over previous
import functools

import jax
import jax.numpy as jnp
from jax import lax
from jax.experimental import pallas as pl
from jax.experimental.pallas import tpu as pltpu
from jax.experimental.pallas import tpu_sc as plsc

NC = 2
NS = 16
L = 16
NW = NC * NS

D = 128
ROWS = 4096 * 200
ROWS_PER_W = ROWS // NW
CHUNK = 100
NCHUNK = ROWS_PER_W // CHUNK
CB = CHUNK * D
NBUF = 4


def _make_sc_permute():
  mesh = plsc.VectorSubcoreMesh(core_axis_name="c", subcore_axis_name="s")

  @functools.partial(
      pl.kernel,
      mesh=mesh,
      out_type=jax.ShapeDtypeStruct((ROWS * D,), jnp.float32),
      scratch_types=(
          [pltpu.VMEM((CB,), jnp.float32) for _ in range(2 * NBUF)]
          + [pltpu.VMEM((D,), jnp.int32)]
          + [pltpu.SemaphoreType.DMA for _ in range(2 * NBUF)]
      ),
      compiler_params=pltpu.CompilerParams(needs_layout_passes=False),
  )
  def permute_kernel(x_hbm, perm_hbm, out_hbm, *scratch):
    ibufs = scratch[:NBUF]
    obufs = scratch[NBUF:2 * NBUF]
    permb = scratch[2 * NBUF]
    isems = scratch[2 * NBUF + 1:2 * NBUF + 1 + NBUF]
    osems = scratch[2 * NBUF + 1 + NBUF:]

    wid = lax.axis_index("s") * NC + lax.axis_index("c")

    pltpu.sync_copy(perm_hbm, permb)
    perm_vecs = [permb[pl.ds(c * L, L)] for c in range(D // L)]

    def permute_chunk(ib, ob):
      @plsc.parallel_loop(0, CHUNK, unroll=4)
      def row_body(r):
        rb = r * D
        for c in range(D // L):
          ob[pl.ds(rb + c * L, L)] = plsc.load_gather(ib, [perm_vecs[c] + rb])

    def chunk_off(g):
      return (g * NW + wid) * CB

    def start_in(g, b):
      pltpu.async_copy(x_hbm.at[pl.ds(chunk_off(g), CB)], ibufs[b], isems[b])

    def start_out(g, b):
      pltpu.async_copy(obufs[b], out_hbm.at[pl.ds(chunk_off(g), CB)],
                       osems[b])

    def wait_in(b):
      pltpu.make_async_copy(x_hbm.at[pl.ds(wid * CB, CB)], ibufs[b],
                            isems[b]).wait()

    def wait_out(b):
      pltpu.make_async_copy(obufs[b], out_hbm.at[pl.ds(wid * CB, CB)],
                            osems[b]).wait()

    for b in range(NBUF):
      start_in(b, b)

    def ring_body(i, carry):
      g = i * NBUF
      for b in range(NBUF):
        @pl.when(i > 0)
        def _():
          wait_out(b)

        wait_in(b)
        permute_chunk(ibufs[b], obufs[b])
        start_out(g + b, b)

        @pl.when(g + b + NBUF < NCHUNK)
        def _():
          start_in(g + b + NBUF, b)
      return carry

    lax.fori_loop(0, NCHUNK // NBUF, ring_body, 0)
    for b in range(NBUF):
      wait_out(b)

  return permute_kernel


_sc_permute = _make_sc_permute()


def kernel(input, permutation):
  x_flat = input.reshape(ROWS * D)
  out_flat = _sc_permute(x_flat, permutation)
  return out_flat.reshape(input.shape)
